# baseline (speedup 1.0000x reference)
import jax
import jax.numpy as jnp
from jax.experimental import pallas as pl
from jax.experimental.pallas import tpu as pltpu

_B = 32
_NCLS = 91
_R = 1000


def _body(cls_ref, tb_ref, pred_ref, out_ref, acc):
    b = pl.program_id(0)

    @pl.when(b == 0)
    def _init():
        acc[0] = 0.0
        acc[1] = 0.0

    for i in range(16):
        cls_row = cls_ref[i]
        tb = tb_ref[i]
        psel = jnp.zeros((4, _R), jnp.float32)
        for c in range(1, _NCLS):
            psel = psel + jnp.where(cls_row == c, pred_ref[i, c], 0.0)
        d = jnp.abs(tb - psel)
        l = jnp.where(d < 1.0, 0.5 * d * d, d - 0.5)
        valid = cls_row > 0
        acc[0] = acc[0] + jnp.sum(jnp.where(valid, l, 0.0))
        acc[1] = acc[1] + 4.0 * jnp.sum(valid.astype(jnp.float32))

    @pl.when(b == _B // 16 - 1)
    def _fin():
        total, count = acc[0], acc[1]
        out_ref[...] = jnp.reshape(
            jnp.where(count > 0, total / jnp.maximum(count, 1.0), 0.0), (1, 1))


def kernel(target_boxes, target_class_ids, pred_boxes):
    cls = target_class_ids.astype(jnp.int32).reshape(_B, 1, _R)
    tb = target_boxes.transpose(0, 2, 1)
    pred = pred_boxes.transpose(0, 2, 3, 1)

    out = pl.pallas_call(
        _body,
        grid=(_B // 16,),
        in_specs=[
            pl.BlockSpec((16, 1, _R), lambda b: (b, 0, 0)),
            pl.BlockSpec((16, 4, _R), lambda b: (b, 0, 0)),
            pl.BlockSpec((16, _NCLS, 4, _R), lambda b: (b, 0, 0, 0)),
        ],
        out_specs=pl.BlockSpec((1, 1), lambda b: (0, 0)),
        out_shape=jax.ShapeDtypeStruct((1, 1), jnp.float32),
        scratch_shapes=[pltpu.SMEM((2,), jnp.float32)],
    )(cls, tb, pred)
    return out[0, 0]


# speedup vs baseline: 1.1260x; 1.1260x over previous
import jax
import jax.numpy as jnp
from jax.experimental import pallas as pl
from jax.experimental.pallas import tpu as pltpu

_B = 32
_NCLS = 91
_R = 1000


def _body(cls_ref, tb_ref, pred_ref, out_ref, acc):
    b = pl.program_id(0)

    @pl.when(b == 0)
    def _init():
        acc[0] = 0.0
        acc[1] = 0.0

    for i in range(8):
        cls_row = cls_ref[i]
        tb = tb_ref[i]
        psel = jnp.zeros((4, _R), jnp.float32)
        for c in range(1, _NCLS):
            psel = psel + jnp.where(cls_row == c, pred_ref[i, c], 0.0)
        d = jnp.abs(tb - psel)
        l = jnp.where(d < 1.0, 0.5 * d * d, d - 0.5)
        valid = cls_row > 0
        acc[0] = acc[0] + jnp.sum(jnp.where(valid, l, 0.0))
        acc[1] = acc[1] + 4.0 * jnp.sum(valid.astype(jnp.float32))

    @pl.when(b == _B // 8 - 1)
    def _fin():
        total, count = acc[0], acc[1]
        out_ref[...] = jnp.reshape(
            jnp.where(count > 0, total / jnp.maximum(count, 1.0), 0.0), (1, 1))


def kernel(target_boxes, target_class_ids, pred_boxes):
    cls = target_class_ids.astype(jnp.int32).reshape(_B, 1, _R)
    tb = target_boxes.transpose(0, 2, 1)
    pred = pred_boxes.transpose(0, 2, 3, 1)

    out = pl.pallas_call(
        _body,
        grid=(_B // 8,),
        in_specs=[
            pl.BlockSpec((8, 1, _R), lambda b: (b, 0, 0)),
            pl.BlockSpec((8, 4, _R), lambda b: (b, 0, 0)),
            pl.BlockSpec((8, _NCLS, 4, _R), lambda b: (b, 0, 0, 0)),
        ],
        out_specs=pl.BlockSpec((1, 1), lambda b: (0, 0)),
        out_shape=jax.ShapeDtypeStruct((1, 1), jnp.float32),
        scratch_shapes=[pltpu.SMEM((2,), jnp.float32)],
    )(cls, tb, pred)
    return out[0, 0]


# speedup vs baseline: 1.1268x; 1.0007x over previous
import jax
import jax.numpy as jnp
from jax.experimental import pallas as pl
from jax.experimental.pallas import tpu as pltpu

_B = 32
_NCLS = 91
_R = 1000
_BB = 8


def _body(cls_ref, tb_ref, pred_ref, out_ref, acc):
    b = pl.program_id(0)

    @pl.when(b == 0)
    def _init():
        acc[0] = 0.0
        acc[1] = 0.0

    for i in range(_BB):
        cls_row = cls_ref[i]
        tb = tb_ref[i]
        psel = jnp.zeros((4, _R), jnp.float32)
        for c in range(1, _NCLS):
            psel = psel + jnp.where(cls_row == c, pred_ref[i, c], 0.0)
        d = jnp.abs(tb - psel)
        l = jnp.where(d < 1.0, 0.5 * d * d, d - 0.5)
        valid = cls_row > 0
        acc[0] = acc[0] + jnp.sum(jnp.where(valid, l, 0.0))
        acc[1] = acc[1] + 4.0 * jnp.sum(valid.astype(jnp.float32))

    @pl.when(b == _B // _BB - 1)
    def _fin():
        total, count = acc[0], acc[1]
        out_ref[...] = jnp.reshape(
            jnp.where(count > 0, total / jnp.maximum(count, 1.0), 0.0), (1, 1))


def kernel(target_boxes, target_class_ids, pred_boxes):
    cls = target_class_ids.astype(jnp.int32).reshape(_B, 1, _R)
    tb = target_boxes.transpose(0, 2, 1)
    pred = pred_boxes.transpose(0, 2, 3, 1)

    out = pl.pallas_call(
        _body,
        grid=(_B // _BB,),
        in_specs=[
            pl.BlockSpec((_BB, 1, _R), lambda b: (b, 0, 0)),
            pl.BlockSpec((_BB, 4, _R), lambda b: (b, 0, 0)),
            pl.BlockSpec((_BB, _NCLS, 4, _R), lambda b: (b, 0, 0, 0)),
        ],
        out_specs=pl.BlockSpec((1, 1), lambda b: (0, 0)),
        out_shape=jax.ShapeDtypeStruct((1, 1), jnp.float32),
        scratch_shapes=[pltpu.SMEM((2,), jnp.float32)],
    )(cls, tb, pred)
    return out[0, 0]
